# baseline (speedup 1.0000x reference)
import math

import jax
import jax.numpy as jnp
from jax import lax
from jax.experimental import pallas as pl
from jax.experimental.pallas import tpu as pltpu
from jax.experimental.pallas import tpu_sc as plsc

_NC = 2
_NS = 16
_LANES = 16
_UNROLL = 8


def _sc_loss(xt, mink, k, log_uniform, inv_scale):
    n_rows, t = xt.shape
    rows_needed = _NC * _NS
    e_per_b = n_rows * k // rows_needed

    mesh = plsc.VectorSubcoreMesh(
        core_axis_name="c", subcore_axis_name="s", num_cores=_NC
    )

    def body(x_hbm, mink_hbm, part_hbm, out_hbm,
             minkv_v, row_a, stage_v, all_v, out_stage, sem_a):
        c = lax.axis_index("c")
        s = lax.axis_index("s")
        w = c * _NS + s
        pltpu.sync_copy(mink_hbm, minkv_v.at[pl.ds(0, k)])

        lane = lax.iota(jnp.int32, _LANES)
        mv = minkv_v[...]

        mj = jnp.sum(jnp.where(lane == w % k, mv, 0))
        row = (w // k) * e_per_b + mj

        cp_a = pltpu.async_copy(x_hbm.at[row], row_a, sem_a)
        cp_a.wait()

        accs = tuple(jnp.zeros((_LANES,), jnp.float32) for _ in range(_UNROLL))

        def step(i, accs):
            base = i * (_LANES * _UNROLL)
            return tuple(
                a + row_a[pl.ds(base + u * _LANES, _LANES)]
                for u, a in enumerate(accs)
            )

        n_it = t // (_LANES * _UNROLL)
        accs = lax.fori_loop(0, n_it, step, accs)
        acc = accs[0]
        for a in accs[1:]:
            acc = acc + a

        stage_v[...] = acc
        pltpu.sync_copy(stage_v, part_hbm.at[c, s])
        plsc.subcore_barrier()

        @pl.when(s == 0)
        def _():
            pltpu.sync_copy(part_hbm.at[c], all_v)
            tot = all_v[0]
            for i in range(1, _NS):
                tot = tot + all_v[i]
            total = jnp.sum(tot)
            res = 0.5 * log_uniform + inv_scale * total
            out_stage[...] = jnp.zeros((_LANES,), jnp.float32) + res
            pltpu.sync_copy(out_stage, out_hbm.at[c])

    run = pl.kernel(
        body,
        mesh=mesh,
        out_type=[
            jax.ShapeDtypeStruct((_NC, _NS, _LANES), jnp.float32),
            jax.ShapeDtypeStruct((_NC, _LANES), jnp.float32),
        ],
        scratch_types=[
            pltpu.VMEM((_LANES,), jnp.int32),
            pltpu.VMEM((t,), jnp.float32),
            pltpu.VMEM((_LANES,), jnp.float32),
            pltpu.VMEM((_NS, _LANES), jnp.float32),
            pltpu.VMEM((_LANES,), jnp.float32),
            pltpu.SemaphoreType.DMA,
        ],
        compiler_params=pltpu.CompilerParams(needs_layout_passes=False),
    )
    return run(xt, mink)


def kernel(log_probs, top_k_indices, min_k_expert_indices, layer_idx):
    b, t, e = log_probs.shape
    n = b * t
    k = min_k_expert_indices.shape[0]

    xt = log_probs.transpose(0, 2, 1).reshape(b * e, t)
    mink = min_k_expert_indices.astype(jnp.int32)

    _, out = _sc_loss(xt, mink, k, math.log(1.0 / k), -1.0 / (k * n))
    return out[0, 0] + out[1, 0]


# speedup vs baseline: 1.2042x; 1.2042x over previous
import math

import jax
import jax.numpy as jnp
from jax import lax
from jax.experimental import pallas as pl
from jax.experimental.pallas import tpu as pltpu
from jax.experimental.pallas import tpu_sc as plsc

_NS = 16
_LANES = 16
_UNROLL = 8


def _sc_loss(xt, mink, k, log_uniform, inv_scale):
    n_rows, t = xt.shape
    rows_needed = 2 * _NS
    e_per_b = n_rows * k // rows_needed

    mesh = plsc.VectorSubcoreMesh(
        core_axis_name="c", subcore_axis_name="s", num_cores=1
    )

    def body(x_hbm, mink_hbm, part_hbm, out_hbm,
             minkv_v, row_a, row_b, stage_v, all_v, out_stage,
             sem_a, sem_b):
        w = lax.axis_index("s")
        pltpu.sync_copy(mink_hbm, minkv_v.at[pl.ds(0, k)])

        lane = lax.iota(jnp.int32, _LANES)
        mv = minkv_v[...]

        def row_of(q):
            mj = jnp.sum(jnp.where(lane == q % k, mv, 0))
            return (q // k) * e_per_b + mj

        cp_a = pltpu.async_copy(x_hbm.at[row_of(w)], row_a, sem_a)
        cp_b = pltpu.async_copy(x_hbm.at[row_of(w + _NS)], row_b, sem_b)
        cp_a.wait()

        accs0 = tuple(jnp.zeros((_LANES,), jnp.float32) for _ in range(_UNROLL))

        def make_step(ref):
            def step(i, accs):
                base = i * (_LANES * _UNROLL)
                return tuple(
                    a + ref[pl.ds(base + u * _LANES, _LANES)]
                    for u, a in enumerate(accs)
                )
            return step

        n_it = t // (_LANES * _UNROLL)
        accs = lax.fori_loop(0, n_it, make_step(row_a), accs0)
        cp_b.wait()
        accs = lax.fori_loop(0, n_it, make_step(row_b), accs)
        acc = accs[0]
        for a in accs[1:]:
            acc = acc + a

        stage_v[...] = acc
        pltpu.sync_copy(stage_v, part_hbm.at[w])
        plsc.subcore_barrier()

        @pl.when(w == 0)
        def _():
            pltpu.sync_copy(part_hbm, all_v)
            tot = all_v[0]
            for i in range(1, _NS):
                tot = tot + all_v[i]
            total = jnp.sum(tot)
            res = log_uniform + inv_scale * total
            out_stage[...] = jnp.zeros((_LANES,), jnp.float32) + res
            pltpu.sync_copy(out_stage, out_hbm)

    run = pl.kernel(
        body,
        mesh=mesh,
        out_type=[
            jax.ShapeDtypeStruct((_NS, _LANES), jnp.float32),
            jax.ShapeDtypeStruct((_LANES,), jnp.float32),
        ],
        scratch_types=[
            pltpu.VMEM((_LANES,), jnp.int32),
            pltpu.VMEM((t,), jnp.float32),
            pltpu.VMEM((t,), jnp.float32),
            pltpu.VMEM((_LANES,), jnp.float32),
            pltpu.VMEM((_NS, _LANES), jnp.float32),
            pltpu.VMEM((_LANES,), jnp.float32),
            pltpu.SemaphoreType.DMA,
            pltpu.SemaphoreType.DMA,
        ],
        compiler_params=pltpu.CompilerParams(needs_layout_passes=False),
    )
    return run(xt, mink)


def kernel(log_probs, top_k_indices, min_k_expert_indices, layer_idx):
    b, t, e = log_probs.shape
    n = b * t
    k = min_k_expert_indices.shape[0]

    xt = log_probs.transpose(0, 2, 1).reshape(b * e, t)
    mink = min_k_expert_indices.astype(jnp.int32)

    _, out = _sc_loss(xt, mink, k, math.log(1.0 / k), -1.0 / (k * n))
    return out[0]
